# baseline (device time: 215432 ns/iter reference)
import jax
import jax.numpy as jnp
from jax import lax
from jax.experimental import pallas as pl
from jax.experimental.pallas import tpu as pltpu

N_DEV = 8
N_HOP = 4
NPC = 4


def kernel(x, w_mat, scale_x, scale_w):
    M, Ks = x.shape
    N = w_mat.shape[1]
    m_per = M // N_DEV
    n_piece = N // NPC

    x8 = x.astype(jnp.float8_e4m3fn)
    w8 = w_mat.astype(jnp.float8_e4m3fn)
    s = (scale_x * scale_w).reshape(1, 1)

    def dotf(a, b):
        return lax.dot_general(
            a, b, (((1,), (0,)), ((), ())),
            preferred_element_type=jnp.float32,
        )

    def body(x_ref, w_ref, s_ref, out_ref, xp_ref, wr_ref, wl_ref,
             x_send_sems, x_recv_sems,
             wr_send_sems, wr_recv_sems, wl_send_sems, wl_recv_sems):
        me = lax.axis_index("i")
        s_val = s_ref[0, 0]

        def gray(i):
            return jnp.where(i < 4, i, 11 - i)

        pos = gray(me)
        right = gray(lax.rem(pos + 1, N_DEV))
        left = gray(lax.rem(pos + N_DEV - 1, N_DEV))

        barrier = pltpu.get_barrier_semaphore()
        for d in range(1, N_DEV):
            peer = lax.rem(me + d, N_DEV)
            pl.semaphore_signal(
                barrier, inc=1,
                device_id=(peer,), device_id_type=pl.DeviceIdType.MESH,
            )
        pl.semaphore_wait(barrier, N_DEV - 1)

        def w_send(buf, src_slot, dst_slot, p, ssems, rsems, dev):
            src = (w_ref.at[:, pl.ds(p * n_piece, n_piece)] if src_slot == 0
                   else buf.at[src_slot - 1, :, pl.ds(p * n_piece, n_piece)])
            return pltpu.make_async_remote_copy(
                src_ref=src,
                dst_ref=buf.at[dst_slot - 1, :, pl.ds(p * n_piece, n_piece)],
                send_sem=ssems.at[src_slot, p],
                recv_sem=rsems.at[dst_slot, p],
                device_id=(dev,),
                device_id_type=pl.DeviceIdType.MESH,
            )

        rdmas = []

        def start(rd):
            rd.start()
            rdmas.append(rd)

        def r_pieces(st):
            return list(range(NPC // 2)) if st == N_HOP else list(range(NPC))

        def l_pieces(st):
            lead = list(range(NPC // 2, NPC))
            return lead if st == N_HOP else lead + list(range(NPC // 2))

        for pr, pll in zip(r_pieces(1), l_pieces(1)):
            start(w_send(wr_ref, 0, 1, pr, wr_send_sems, wr_recv_sems,
                         right))
            start(w_send(wl_ref, 0, 1, pll, wl_send_sems, wl_recv_sems,
                         left))

        for d in range(1, N_DEV):
            dest = gray(lax.rem(pos + d, N_DEV))
            start(pltpu.make_async_remote_copy(
                src_ref=x_ref.at[pl.ds(dest * m_per, m_per), :],
                dst_ref=xp_ref.at[d],
                send_sem=x_send_sems.at[d],
                recv_sem=x_recv_sems.at[d],
                device_id=(dest,),
                device_id_type=pl.DeviceIdType.MESH,
            ))

        xo = x_ref[pl.ds(me * m_per, m_per), :]
        out_ref[...] = dotf(xo, w_ref[...]) * s_val

        def x_wait(slot):
            pltpu.make_async_remote_copy(
                src_ref=xp_ref.at[slot],
                dst_ref=xp_ref.at[slot],
                send_sem=x_send_sems.at[slot],
                recv_sem=x_recv_sems.at[slot],
                device_id=(right,),
                device_id_type=pl.DeviceIdType.MESH,
            ).wait_recv()

        def stream_step(buf, st, pieces, nxt_pieces, ssems, rsems, dev,
                        xp_slot, wait_x=True):
            for p in pieces:
                w_send(buf, st, st, p, ssems, rsems, dev).wait_recv()
                if st < N_HOP and p in nxt_pieces:
                    start(w_send(buf, st, st + 1, p, ssems, rsems, dev))
                if wait_x and p == pieces[0]:
                    x_wait(xp_slot)
                cols = pl.ds(p * n_piece, n_piece)
                out_ref[:, cols] = out_ref[:, cols] + dotf(
                    xp_ref[xp_slot], buf[st - 1, :, cols]
                ) * s_val

        for st in range(1, N_HOP + 1):
            stream_step(wr_ref, st, r_pieces(st), r_pieces(st + 1),
                        wr_send_sems, wr_recv_sems, right, st)
            stream_step(wl_ref, st, l_pieces(st), l_pieces(st + 1),
                        wl_send_sems, wl_recv_sems, left, N_DEV - st,
                        wait_x=(st < N_HOP))

        for rd in rdmas:
            rd.wait_send()

    return pl.pallas_call(
        body,
        out_shape=jax.ShapeDtypeStruct((m_per, N), jnp.float32),
        in_specs=[
            pl.BlockSpec(memory_space=pltpu.VMEM),
            pl.BlockSpec(memory_space=pltpu.VMEM),
            pl.BlockSpec(memory_space=pltpu.SMEM),
        ],
        out_specs=pl.BlockSpec(memory_space=pltpu.VMEM),
        scratch_shapes=[
            pltpu.VMEM((N_DEV, m_per, Ks), jnp.float8_e4m3fn),
            pltpu.VMEM((N_HOP, Ks, N), jnp.float8_e4m3fn),
            pltpu.VMEM((N_HOP, Ks, N), jnp.float8_e4m3fn),
            pltpu.SemaphoreType.DMA((N_DEV,)),
            pltpu.SemaphoreType.DMA((N_DEV,)),
            pltpu.SemaphoreType.DMA((N_HOP + 1, NPC)),
            pltpu.SemaphoreType.DMA((N_HOP + 1, NPC)),
            pltpu.SemaphoreType.DMA((N_HOP + 1, NPC)),
            pltpu.SemaphoreType.DMA((N_HOP + 1, NPC)),
        ],
        compiler_params=pltpu.CompilerParams(
            collective_id=0,
            vmem_limit_bytes=100 * 1024 * 1024,
        ),
    )(x8, w8, s)


# device time: 212939 ns/iter; 1.0117x vs baseline; 1.0117x over previous
import jax
import jax.numpy as jnp
from jax import lax
from jax.experimental import pallas as pl
from jax.experimental.pallas import tpu as pltpu

N_DEV = 8
N_HOP = 4
NPC = 4


def kernel(x, w_mat, scale_x, scale_w):
    M, Ks = x.shape
    N = w_mat.shape[1]
    m_per = M // N_DEV
    n_piece = N // NPC

    def cast_body(x_ref, w_ref, x8_ref, w8_ref):
        x8_ref[...] = x_ref[...].astype(jnp.float8_e4m3fn)
        w8_ref[...] = w_ref[...].astype(jnp.float8_e4m3fn)

    x8, w8 = pl.pallas_call(
        cast_body,
        out_shape=(
            jax.ShapeDtypeStruct((M, Ks), jnp.float8_e4m3fn),
            jax.ShapeDtypeStruct((Ks, N), jnp.float8_e4m3fn),
        ),
        in_specs=[
            pl.BlockSpec(memory_space=pltpu.VMEM),
            pl.BlockSpec(memory_space=pltpu.VMEM),
        ],
        out_specs=(
            pl.BlockSpec(memory_space=pltpu.VMEM),
            pl.BlockSpec(memory_space=pltpu.VMEM),
        ),
        compiler_params=pltpu.CompilerParams(
            vmem_limit_bytes=100 * 1024 * 1024,
        ),
    )(x, w_mat)
    s = (scale_x * scale_w).reshape(1, 1)

    def dotf(a, b):
        return lax.dot_general(
            a, b, (((1,), (0,)), ((), ())),
            preferred_element_type=jnp.float32,
        )

    def body(x_ref, w_ref, s_ref, out_ref, xp_ref, wr_ref, wl_ref,
             x_send_sems, x_recv_sems,
             wr_send_sems, wr_recv_sems, wl_send_sems, wl_recv_sems):
        me = lax.axis_index("i")
        s_val = s_ref[0, 0]

        def gray(i):
            return jnp.where(i < 4, i, 11 - i)

        pos = gray(me)
        right = gray(lax.rem(pos + 1, N_DEV))
        left = gray(lax.rem(pos + N_DEV - 1, N_DEV))

        barrier = pltpu.get_barrier_semaphore()
        for d in range(1, N_DEV):
            peer = lax.rem(me + d, N_DEV)
            pl.semaphore_signal(
                barrier, inc=1,
                device_id=(peer,), device_id_type=pl.DeviceIdType.MESH,
            )
        pl.semaphore_wait(barrier, N_DEV - 1)

        def w_send(buf, src_slot, dst_slot, p, ssems, rsems, dev):
            src = (w_ref.at[:, pl.ds(p * n_piece, n_piece)] if src_slot == 0
                   else buf.at[src_slot - 1, :, pl.ds(p * n_piece, n_piece)])
            return pltpu.make_async_remote_copy(
                src_ref=src,
                dst_ref=buf.at[dst_slot - 1, :, pl.ds(p * n_piece, n_piece)],
                send_sem=ssems.at[src_slot, p],
                recv_sem=rsems.at[dst_slot, p],
                device_id=(dev,),
                device_id_type=pl.DeviceIdType.MESH,
            )

        rdmas = []

        def start(rd):
            rd.start()
            rdmas.append(rd)

        def r_pieces(st):
            return list(range(NPC // 2)) if st == N_HOP else list(range(NPC))

        def l_pieces(st):
            lead = list(range(NPC // 2, NPC))
            return lead if st == N_HOP else lead + list(range(NPC // 2))

        for pr, pll in zip(r_pieces(1), l_pieces(1)):
            start(w_send(wr_ref, 0, 1, pr, wr_send_sems, wr_recv_sems,
                         right))
            start(w_send(wl_ref, 0, 1, pll, wl_send_sems, wl_recv_sems,
                         left))

        for d in range(1, N_DEV):
            dest = gray(lax.rem(pos + d, N_DEV))
            start(pltpu.make_async_remote_copy(
                src_ref=x_ref.at[pl.ds(dest * m_per, m_per), :],
                dst_ref=xp_ref.at[d],
                send_sem=x_send_sems.at[d],
                recv_sem=x_recv_sems.at[d],
                device_id=(dest,),
                device_id_type=pl.DeviceIdType.MESH,
            ))

        xo = x_ref[pl.ds(me * m_per, m_per), :]
        out_ref[...] = dotf(xo, w_ref[...]) * s_val

        def x_wait(slot):
            pltpu.make_async_remote_copy(
                src_ref=xp_ref.at[slot],
                dst_ref=xp_ref.at[slot],
                send_sem=x_send_sems.at[slot],
                recv_sem=x_recv_sems.at[slot],
                device_id=(right,),
                device_id_type=pl.DeviceIdType.MESH,
            ).wait_recv()

        def stream_step(buf, st, pieces, nxt_pieces, ssems, rsems, dev,
                        xp_slot, wait_x=True):
            for p in pieces:
                w_send(buf, st, st, p, ssems, rsems, dev).wait_recv()
                if st < N_HOP and p in nxt_pieces:
                    start(w_send(buf, st, st + 1, p, ssems, rsems, dev))
                if wait_x and p == pieces[0]:
                    x_wait(xp_slot)
                cols = pl.ds(p * n_piece, n_piece)
                out_ref[:, cols] = out_ref[:, cols] + dotf(
                    xp_ref[xp_slot], buf[st - 1, :, cols]
                ) * s_val

        for st in range(1, N_HOP + 1):
            stream_step(wr_ref, st, r_pieces(st), r_pieces(st + 1),
                        wr_send_sems, wr_recv_sems, right, st)
            stream_step(wl_ref, st, l_pieces(st), l_pieces(st + 1),
                        wl_send_sems, wl_recv_sems, left, N_DEV - st,
                        wait_x=(st < N_HOP))

        for rd in rdmas:
            rd.wait_send()

    return pl.pallas_call(
        body,
        out_shape=jax.ShapeDtypeStruct((m_per, N), jnp.float32),
        in_specs=[
            pl.BlockSpec(memory_space=pltpu.VMEM),
            pl.BlockSpec(memory_space=pltpu.VMEM),
            pl.BlockSpec(memory_space=pltpu.SMEM),
        ],
        out_specs=pl.BlockSpec(memory_space=pltpu.VMEM),
        scratch_shapes=[
            pltpu.VMEM((N_DEV, m_per, Ks), jnp.float8_e4m3fn),
            pltpu.VMEM((N_HOP, Ks, N), jnp.float8_e4m3fn),
            pltpu.VMEM((N_HOP, Ks, N), jnp.float8_e4m3fn),
            pltpu.SemaphoreType.DMA((N_DEV,)),
            pltpu.SemaphoreType.DMA((N_DEV,)),
            pltpu.SemaphoreType.DMA((N_HOP + 1, NPC)),
            pltpu.SemaphoreType.DMA((N_HOP + 1, NPC)),
            pltpu.SemaphoreType.DMA((N_HOP + 1, NPC)),
            pltpu.SemaphoreType.DMA((N_HOP + 1, NPC)),
        ],
        compiler_params=pltpu.CompilerParams(
            collective_id=0,
            vmem_limit_bytes=100 * 1024 * 1024,
        ),
    )(x8, w8, s)
